# Initial kernel scaffold; baseline (speedup 1.0000x reference)
import jax
import jax.numpy as jnp
from jax.experimental import pallas as pl


def kernel(u, v, user_emb, item_emb, W1, b1, W2, b2):
    raise NotImplementedError("write your pallas kernel here")


# speedup vs baseline: 2.7133x; 2.7133x over previous
import functools

import jax
import jax.numpy as jnp
from jax import lax
from jax.experimental import pallas as pl
from jax.experimental.pallas import tpu as pltpu
from jax.experimental.pallas import tpu_sc as plsc


@functools.partial(jax.jit, static_argnums=(4, 5, 6))
def _sc_gather(user_emb, item_emb, u2, v2, B, D, CHUNK):
    info = plsc.get_sparse_core_info()
    NC, NS = info.num_cores, info.num_subcores
    NW = NC * NS
    b_per_w = B // NW
    n_chunks = b_per_w // CHUNK

    mesh = plsc.VectorSubcoreMesh(core_axis_name="c", subcore_axis_name="s")

    @functools.partial(
        pl.kernel,
        mesh=mesh,
        out_type=[
            jax.ShapeDtypeStruct((B, D), jnp.float32),
            jax.ShapeDtypeStruct((B, D), jnp.float32),
        ],
        scratch_types=[
            pltpu.VMEM((n_chunks, CHUNK), jnp.int32),
            pltpu.VMEM((n_chunks, CHUNK), jnp.int32),
            pltpu.VMEM((CHUNK, D), jnp.float32),
            pltpu.VMEM((CHUNK, D), jnp.float32),
            pltpu.SemaphoreType.DMA,
            pltpu.SemaphoreType.DMA,
        ],
    )
    def k(uemb_hbm, vemb_hbm, u_hbm, v_hbm, U_out, V_out,
          uidx, vidx, rows_a, rows_b, sem_a, sem_b):
        wid = lax.axis_index("s") * NC + lax.axis_index("c")
        base = wid * b_per_w
        pltpu.sync_copy(u_hbm.at[wid], uidx)
        pltpu.sync_copy(v_hbm.at[wid], vidx)
        for j in range(n_chunks):
            cu = pltpu.async_copy(uemb_hbm.at[uidx.at[j]], rows_a, sem_a)
            cv = pltpu.async_copy(vemb_hbm.at[vidx.at[j]], rows_b, sem_b)
            cu.wait()
            pltpu.sync_copy(rows_a, U_out.at[pl.ds(base + j * CHUNK, CHUNK)])
            cv.wait()
            pltpu.sync_copy(rows_b, V_out.at[pl.ds(base + j * CHUNK, CHUNK)])

    return k(user_emb, item_emb, u2, v2)


def _mlp_body(U_ref, V_ref, W1uT_ref, W1vT_ref, b1_ref, W2T_ref, b2_ref, o_ref):
    xu = jnp.maximum(U_ref[...], 0.0)
    xv = jnp.maximum(V_ref[...], 0.0)
    h = jnp.dot(xu, W1uT_ref[...], preferred_element_type=jnp.float32)
    h = h + jnp.dot(xv, W1vT_ref[...], preferred_element_type=jnp.float32)
    h = jnp.maximum(h + b1_ref[...], 0.0)
    o_ref[...] = jnp.dot(h, W2T_ref[...], preferred_element_type=jnp.float32) + b2_ref[...]


@functools.partial(jax.jit, static_argnums=(7,))
def _tc_mlp(U, V, W1uT, W1vT, b1, W2T, b2, bm):
    B, D = U.shape
    H = W1uT.shape[1]
    grid = (B // bm,)
    return pl.pallas_call(
        _mlp_body,
        grid=grid,
        in_specs=[
            pl.BlockSpec((bm, D), lambda i: (i, 0)),
            pl.BlockSpec((bm, D), lambda i: (i, 0)),
            pl.BlockSpec((D, H), lambda i: (0, 0)),
            pl.BlockSpec((D, H), lambda i: (0, 0)),
            pl.BlockSpec((1, H), lambda i: (0, 0)),
            pl.BlockSpec((H, 1), lambda i: (0, 0)),
            pl.BlockSpec((1, 1), lambda i: (0, 0)),
        ],
        out_specs=pl.BlockSpec((bm, 1), lambda i: (i, 0)),
        out_shape=jax.ShapeDtypeStruct((B, 1), jnp.float32),
    )(U, V, W1uT, W1vT, b1, W2T, b2)


def kernel(u, v, user_emb, item_emb, W1, b1, W2, b2):
    B = u.shape[0]
    D = user_emb.shape[1]
    H = W1.shape[0]
    info = plsc.get_sparse_core_info()
    NW = info.num_cores * info.num_subcores
    CHUNK = 128
    b_per_w = B // NW
    n_chunks = b_per_w // CHUNK

    u2 = u.astype(jnp.int32).reshape(NW, n_chunks, CHUNK)
    v2 = v.astype(jnp.int32).reshape(NW, n_chunks, CHUNK)
    U, V = _sc_gather(user_emb, item_emb, u2, v2, B, D, CHUNK)

    W1uT = W1[:, :D].T
    W1vT = W1[:, D:].T
    return _tc_mlp(U, V, W1uT, W1vT, b1.reshape(1, H), W2.T, b2.reshape(1, 1),
                   2048)


# speedup vs baseline: 2.9942x; 1.1035x over previous
import functools

import jax
import jax.numpy as jnp
from jax import lax
from jax.experimental import pallas as pl
from jax.experimental.pallas import tpu as pltpu
from jax.experimental.pallas import tpu_sc as plsc


@functools.partial(jax.jit, static_argnums=(4, 5, 6))
def _sc_gather(user_emb, item_emb, u2, v2, B, D, CHUNK):
    info = plsc.get_sparse_core_info()
    NC, NS = info.num_cores, info.num_subcores
    NW = NC * NS
    b_per_w = B // NW
    n_chunks = b_per_w // CHUNK

    mesh = plsc.VectorSubcoreMesh(core_axis_name="c", subcore_axis_name="s")

    @functools.partial(
        pl.kernel,
        mesh=mesh,
        out_type=[
            jax.ShapeDtypeStruct((B, D), jnp.float32),
            jax.ShapeDtypeStruct((B, D), jnp.float32),
        ],
        scratch_types=[
            pltpu.VMEM((n_chunks, CHUNK), jnp.int32),
            pltpu.VMEM((n_chunks, CHUNK), jnp.int32),
            pltpu.VMEM((CHUNK, D), jnp.float32),
            pltpu.VMEM((CHUNK, D), jnp.float32),
            pltpu.VMEM((CHUNK, D), jnp.float32),
            pltpu.VMEM((CHUNK, D), jnp.float32),
            pltpu.SemaphoreType.DMA,
            pltpu.SemaphoreType.DMA,
            pltpu.SemaphoreType.DMA,
            pltpu.SemaphoreType.DMA,
        ],
    )
    def k(uemb_hbm, vemb_hbm, u_hbm, v_hbm, U_out, V_out,
          uidx, vidx, ua, ub, va, vb, sua, sub_, sva, svb):
        wid = lax.axis_index("s") * NC + lax.axis_index("c")
        base = wid * b_per_w
        ubufs = (ua, ub)
        vbufs = (va, vb)
        usems = (sua, sub_)
        vsems = (sva, svb)
        pltpu.sync_copy(u_hbm.at[wid], uidx)
        pltpu.sync_copy(v_hbm.at[wid], vidx)
        cu = [None] * n_chunks
        cv = [None] * n_chunks
        cu[0] = pltpu.async_copy(uemb_hbm.at[uidx.at[0]], ubufs[0], usems[0])
        cv[0] = pltpu.async_copy(vemb_hbm.at[vidx.at[0]], vbufs[0], vsems[0])
        for j in range(n_chunks):
            nxt = (j + 1) % 2
            if j + 1 < n_chunks:
                cu[j + 1] = pltpu.async_copy(
                    uemb_hbm.at[uidx.at[j + 1]], ubufs[nxt], usems[nxt])
                cv[j + 1] = pltpu.async_copy(
                    vemb_hbm.at[vidx.at[j + 1]], vbufs[nxt], vsems[nxt])
            cur = j % 2
            cu[j].wait()
            pltpu.sync_copy(ubufs[cur], U_out.at[pl.ds(base + j * CHUNK, CHUNK)])
            cv[j].wait()
            pltpu.sync_copy(vbufs[cur], V_out.at[pl.ds(base + j * CHUNK, CHUNK)])

    return k(user_emb, item_emb, u2, v2)


_DN_RHS_T = (((1,), (1,)), ((), ()))


def _mlp_body(U_ref, V_ref, W1_ref, b1_ref, W2_ref, b2_ref, o_ref):
    D = U_ref.shape[1]
    xu = jnp.maximum(U_ref[...], 0.0)
    xv = jnp.maximum(V_ref[...], 0.0)
    h = lax.dot_general(xu, W1_ref[:, :D], _DN_RHS_T,
                        preferred_element_type=jnp.float32)
    h = h + lax.dot_general(xv, W1_ref[:, D:], _DN_RHS_T,
                            preferred_element_type=jnp.float32)
    h = jnp.maximum(h + b1_ref[...], 0.0)
    o_ref[...] = lax.dot_general(W2_ref[...], h, _DN_RHS_T,
                                 preferred_element_type=jnp.float32) + b2_ref[...]


@functools.partial(jax.jit, static_argnums=(6,))
def _tc_mlp(U, V, W1, b1, W2, b2, bm):
    B, D = U.shape
    H = W1.shape[0]
    grid = (B // bm,)
    return pl.pallas_call(
        _mlp_body,
        grid=grid,
        in_specs=[
            pl.BlockSpec((bm, D), lambda i: (i, 0)),
            pl.BlockSpec((bm, D), lambda i: (i, 0)),
            pl.BlockSpec((H, 2 * D), lambda i: (0, 0)),
            pl.BlockSpec((1, H), lambda i: (0, 0)),
            pl.BlockSpec((1, H), lambda i: (0, 0)),
            pl.BlockSpec((1, 1), lambda i: (0, 0)),
        ],
        out_specs=pl.BlockSpec((1, bm), lambda i: (0, i)),
        out_shape=jax.ShapeDtypeStruct((1, B), jnp.float32),
    )(U, V, W1, b1, W2, b2)


def kernel(u, v, user_emb, item_emb, W1, b1, W2, b2):
    B = u.shape[0]
    D = user_emb.shape[1]
    H = W1.shape[0]
    info = plsc.get_sparse_core_info()
    NW = info.num_cores * info.num_subcores
    CHUNK = 128
    b_per_w = B // NW
    n_chunks = b_per_w // CHUNK

    u2 = u.astype(jnp.int32).reshape(NW, n_chunks, CHUNK)
    v2 = v.astype(jnp.int32).reshape(NW, n_chunks, CHUNK)
    U, V = _sc_gather(user_emb, item_emb, u2, v2, B, D, CHUNK)

    row = _tc_mlp(U, V, W1, b1.reshape(1, H), W2, b2.reshape(1, 1), 1024)
    return row.reshape(B, 1)


# speedup vs baseline: 3.2614x; 1.0892x over previous
import functools

import jax
import jax.numpy as jnp
from jax import lax
from jax.experimental import pallas as pl
from jax.experimental.pallas import tpu as pltpu
from jax.experimental.pallas import tpu_sc as plsc


@functools.partial(jax.jit, static_argnums=(4, 5, 6))
def _sc_gather(user_emb, item_emb, u2, v2, B, D, CHUNK):
    info = plsc.get_sparse_core_info()
    NC, NS = info.num_cores, info.num_subcores
    NW = NC * NS
    b_per_w = B // NW
    n_chunks = b_per_w // CHUNK

    mesh = plsc.VectorSubcoreMesh(core_axis_name="c", subcore_axis_name="s")

    @functools.partial(
        pl.kernel,
        mesh=mesh,
        out_type=[
            jax.ShapeDtypeStruct((B, D), jnp.float32),
            jax.ShapeDtypeStruct((B, D), jnp.float32),
        ],
        scratch_types=[
            pltpu.VMEM((n_chunks, CHUNK), jnp.int32),
            pltpu.VMEM((n_chunks, CHUNK), jnp.int32),
            pltpu.VMEM((CHUNK, D), jnp.float32),
            pltpu.VMEM((CHUNK, D), jnp.float32),
            pltpu.VMEM((CHUNK, D), jnp.float32),
            pltpu.VMEM((CHUNK, D), jnp.float32),
            pltpu.SemaphoreType.DMA,
            pltpu.SemaphoreType.DMA,
            pltpu.SemaphoreType.DMA,
            pltpu.SemaphoreType.DMA,
        ],
    )
    def k(uemb_hbm, vemb_hbm, u_hbm, v_hbm, U_out, V_out,
          uidx, vidx, ua, ub, va, vb, sua, sub_, sva, svb):
        wid = lax.axis_index("s") * NC + lax.axis_index("c")
        base = wid * b_per_w
        ubufs = (ua, ub)
        vbufs = (va, vb)
        usems = (sua, sub_)
        vsems = (sva, svb)
        pltpu.sync_copy(u_hbm.at[wid], uidx)
        pltpu.sync_copy(v_hbm.at[wid], vidx)
        cu = [None] * n_chunks
        cv = [None] * n_chunks
        cu[0] = pltpu.async_copy(uemb_hbm.at[uidx.at[0]], ubufs[0], usems[0])
        cv[0] = pltpu.async_copy(vemb_hbm.at[vidx.at[0]], vbufs[0], vsems[0])
        for j in range(n_chunks):
            nxt = (j + 1) % 2
            if j + 1 < n_chunks:
                cu[j + 1] = pltpu.async_copy(
                    uemb_hbm.at[uidx.at[j + 1]], ubufs[nxt], usems[nxt])
                cv[j + 1] = pltpu.async_copy(
                    vemb_hbm.at[vidx.at[j + 1]], vbufs[nxt], vsems[nxt])
            cur = j % 2
            cu[j].wait()
            pltpu.sync_copy(ubufs[cur], U_out.at[pl.ds(base + j * CHUNK, CHUNK)])
            cv[j].wait()
            pltpu.sync_copy(vbufs[cur], V_out.at[pl.ds(base + j * CHUNK, CHUNK)])

    return k(user_emb, item_emb, u2, v2)


_DN_RHS_T = (((1,), (1,)), ((), ()))


def _mlp_body(U_ref, V_ref, W1_ref, b1_ref, W2_ref, b2_ref, o_ref):
    xu = jnp.maximum(U_ref[...], 0.0)
    xv = jnp.maximum(V_ref[...], 0.0)
    x = jnp.concatenate([xu, xv], axis=1)
    h = lax.dot_general(x, W1_ref[...], _DN_RHS_T,
                        preferred_element_type=jnp.float32)
    h = jnp.maximum(h + b1_ref[...], 0.0)
    o_ref[...] = lax.dot_general(W2_ref[...], h, _DN_RHS_T,
                                 preferred_element_type=jnp.float32) + b2_ref[...]


@functools.partial(jax.jit, static_argnums=(6,))
def _tc_mlp(U, V, W1, b1, W2, b2, bm):
    B, D = U.shape
    H = W1.shape[0]
    grid = (B // bm,)
    return pl.pallas_call(
        _mlp_body,
        grid=grid,
        in_specs=[
            pl.BlockSpec((bm, D), lambda i: (i, 0)),
            pl.BlockSpec((bm, D), lambda i: (i, 0)),
            pl.BlockSpec((H, 2 * D), lambda i: (0, 0)),
            pl.BlockSpec((1, H), lambda i: (0, 0)),
            pl.BlockSpec((1, H), lambda i: (0, 0)),
            pl.BlockSpec((1, 1), lambda i: (0, 0)),
        ],
        out_specs=pl.BlockSpec((1, bm), lambda i: (0, i)),
        out_shape=jax.ShapeDtypeStruct((1, B), jnp.float32),
    )(U, V, W1, b1, W2, b2)


def kernel(u, v, user_emb, item_emb, W1, b1, W2, b2):
    B = u.shape[0]
    D = user_emb.shape[1]
    H = W1.shape[0]
    info = plsc.get_sparse_core_info()
    NW = info.num_cores * info.num_subcores
    CHUNK = 128
    N_SLICES = 2
    Bs = B // N_SLICES
    n_chunks = Bs // NW // CHUNK

    ui = u.astype(jnp.int32).reshape(N_SLICES, NW, n_chunks, CHUNK)
    vi = v.astype(jnp.int32).reshape(N_SLICES, NW, n_chunks, CHUNK)
    b1r = b1.reshape(1, H)
    b2r = b2.reshape(1, 1)

    gathered = [
        _sc_gather(user_emb, item_emb, ui[s], vi[s], Bs, D, CHUNK)
        for s in range(N_SLICES)
    ]
    rows = [
        _tc_mlp(U, V, W1, b1r, W2, b2r, 2048) for (U, V) in gathered
    ]
    return jnp.concatenate(rows, axis=1).reshape(B, 1)
